# Initial kernel scaffold; baseline (speedup 1.0000x reference)
import functools

import jax
import jax.numpy as jnp
from jax import lax
from jax.experimental import pallas as pl
from jax.experimental.pallas import tpu as pltpu
from jax.experimental.pallas import tpu_sc as plsc

N = 10000
E = 320000
D = 128
K_HOPS = 4
ALPHA = 0.5
BETA = 1.0

NC = 2
NS = 16
L = 16

DH = D // NC
N_PAD = 10240
R_T = N_PAD // NS
N_RCH = R_T // 128
E_T = E // NS
C_T = (E_T + 127) // 128
E_T_PAD = C_T * 128
PAD_SRC = 0
PAD_DST = N + 100


def _body(x_r, src_r, dst_r, zeros2d, zeros1d, out_hbm,
          sidx, didx, gbuf, zbuf, abuf, outbuf, degbuf, onesb,
          h_sh, acc_sh, deg_sh):
  cid = lax.axis_index("c")
  sid = lax.axis_index("s")
  tile_base = pl.multiple_of(sid * R_T, 128)

  pltpu.sync_copy(src_r.at[sid], sidx)
  pltpu.sync_copy(dst_r.at[sid], didx)
  pltpu.sync_copy(zeros2d, zbuf)

  pltpu.sync_copy(x_r.at[cid, pl.ds(tile_base, R_T)], outbuf)
  pltpu.sync_copy(outbuf, h_sh.at[pl.ds(tile_base, R_T)])
  pltpu.sync_copy(zeros1d, deg_sh.at[pl.ds(tile_base, R_T)])

  def _ones(i, _):
    base = pl.multiple_of(i * L, L)
    onesb[pl.ds(base, L)] = jnp.ones((L,), jnp.float32)
    return 0
  lax.fori_loop(0, 128 // L, _ones, 0)

  plsc.subcore_barrier()

  def _deg(j, _):
    pltpu.sync_copy(onesb, deg_sh.at[didx.at[j]], add=True)
    return 0
  lax.fori_loop(0, C_T, _deg, 0)
  plsc.subcore_barrier()

  pltpu.sync_copy(deg_sh.at[pl.ds(tile_base, R_T)], degbuf)
  def _inv(i, _):
    base = pl.multiple_of(i * L, L)
    v = degbuf[pl.ds(base, L)]
    degbuf[pl.ds(base, L)] = jnp.where(
        v > 0.0, 1.0 / jnp.maximum(v, 1.0), 0.0)
    return 0
  lax.fori_loop(0, R_T // L, _inv, 0)

  for k in range(K_HOPS):
    w = ALPHA ** (k + 1)

    for ch in range(N_RCH):
      row0 = pl.multiple_of(tile_base + ch * 128, 128)
      pltpu.sync_copy(zbuf, acc_sh.at[pl.ds(row0, 128)])
    plsc.subcore_barrier()

    def _edge(j, _):
      pltpu.sync_copy(h_sh.at[sidx.at[j]], gbuf)
      pltpu.sync_copy(gbuf, acc_sh.at[didx.at[j]], add=True)
      return 0
    lax.fori_loop(0, C_T, _edge, 0)
    plsc.subcore_barrier()

    for ch in range(N_RCH):
      row0 = pl.multiple_of(tile_base + ch * 128, 128)
      pltpu.sync_copy(acc_sh.at[pl.ds(row0, 128)], abuf)

      def _norm(r, _, ch=ch):
        lrow = ch * 128 + r
        bc = plsc.load_gather(degbuf, [jnp.full((L,), lrow, jnp.int32)])
        for f in range(DH // L):
          a = abuf[r, pl.ds(f * L, L)]
          h16 = a * bc
          abuf[r, pl.ds(f * L, L)] = h16
          outbuf[lrow, pl.ds(f * L, L)] = (
              outbuf[lrow, pl.ds(f * L, L)] + w * h16)
        return 0
      lax.fori_loop(0, 128, _norm, 0)

      if k != K_HOPS - 1:
        pltpu.sync_copy(abuf, h_sh.at[pl.ds(row0, 128)])
    if k != K_HOPS - 1:
      plsc.subcore_barrier()

  pltpu.sync_copy(outbuf, out_hbm.at[cid, pl.ds(tile_base, R_T)])


@jax.jit
def kernel(x, edge_index):
  x_p = jnp.pad(x, ((0, N_PAD - N), (0, 0)))
  x_r = x_p.reshape(N_PAD, NC, DH).transpose(1, 0, 2)

  src = edge_index[0].reshape(NS, E_T)
  dst = edge_index[1].reshape(NS, E_T)
  src_r = jnp.pad(src, ((0, 0), (0, E_T_PAD - E_T)),
                  constant_values=PAD_SRC).reshape(NS, C_T, 128)
  dst_r = jnp.pad(dst, ((0, 0), (0, E_T_PAD - E_T)),
                  constant_values=PAD_DST).reshape(NS, C_T, 128)

  zeros2d = jnp.zeros((128, DH), jnp.float32)
  zeros1d = jnp.zeros((R_T,), jnp.float32)

  mesh = plsc.VectorSubcoreMesh(core_axis_name="c", subcore_axis_name="s",
                                num_cores=NC, num_subcores=NS)
  out_r = pl.kernel(
      _body,
      out_type=jax.ShapeDtypeStruct((NC, N_PAD, DH), jnp.float32),
      mesh=mesh,
      scratch_types=[
          pltpu.VMEM((C_T, 128), jnp.int32),
          pltpu.VMEM((C_T, 128), jnp.int32),
          pltpu.VMEM((128, DH), jnp.float32),
          pltpu.VMEM((128, DH), jnp.float32),
          pltpu.VMEM((128, DH), jnp.float32),
          pltpu.VMEM((R_T, DH), jnp.float32),
          pltpu.VMEM((R_T,), jnp.float32),
          pltpu.VMEM((128,), jnp.float32),
          pltpu.VMEM_SHARED((N_PAD, DH), jnp.float32),
          pltpu.VMEM_SHARED((N_PAD, DH), jnp.float32),
          pltpu.VMEM_SHARED((N_PAD,), jnp.float32),
      ],
  )(x_r, src_r, dst_r, zeros2d, zeros1d)

  out = out_r.transpose(1, 0, 2).reshape(N_PAD, D)
  return out[:N]


# speedup vs baseline: 5.9453x; 5.9453x over previous
import jax
import jax.numpy as jnp
from jax import lax
from jax.experimental import pallas as pl
from jax.experimental.pallas import tpu as pltpu
from jax.experimental.pallas import tpu_sc as plsc

N = 10000
E = 320000
D = 128
K_HOPS = 4
ALPHA = 0.5
BETA = 1.0

NC = 2
NS = 16
L = 16

DH = D // NC
N_PAD = 10240
R_T = N_PAD // NS
N_RCH = R_T // 128
E_T = E // NS
B_CH = 8
N_BLK = 20
C_T = B_CH * N_BLK
E_T_PAD = C_T * 128
PAD_SRC = 0
PAD_DST = N + 100


def _body(x_r, src_r, dst_r, zeros2d, zeros1d, out_hbm,
          sidxb, didxb, wbuf0, wbuf1, zbuf, degbuf, onesb,
          h_sh, acc_sh, deg_sh):
  cid = lax.axis_index("c")
  sid = lax.axis_index("s")
  tile_base = pl.multiple_of(sid * R_T, 128)

  pltpu.sync_copy(zeros2d, zbuf)
  for ch in range(N_RCH):
    row0 = pl.multiple_of(tile_base + ch * 128, 128)
    pltpu.sync_copy(x_r.at[cid, pl.ds(row0, 128)], wbuf0)
    pltpu.sync_copy(wbuf0, h_sh.at[pl.ds(row0, 128)])
    pltpu.sync_copy(wbuf0, out_hbm.at[cid, pl.ds(row0, 128)])
  pltpu.sync_copy(zeros1d, deg_sh.at[pl.ds(tile_base, R_T)])

  def _ones(i, _):
    base = pl.multiple_of(i * L, L)
    onesb[pl.ds(base, L)] = jnp.ones((L,), jnp.float32)
    return 0
  lax.fori_loop(0, 128 // L, _ones, 0)

  plsc.subcore_barrier()

  def _deg(b, _):
    pltpu.sync_copy(dst_r.at[sid, pl.ds(pl.multiple_of(b * B_CH, B_CH), B_CH)],
                    didxb)
    for jj in range(B_CH):
      pltpu.sync_copy(onesb, deg_sh.at[didxb.at[jj]], add=True)
    return 0
  lax.fori_loop(0, N_BLK, _deg, 0)
  plsc.subcore_barrier()

  pltpu.sync_copy(deg_sh.at[pl.ds(tile_base, R_T)], degbuf)
  def _inv(i, _):
    base = pl.multiple_of(i * L, L)
    v = degbuf[pl.ds(base, L)]
    degbuf[pl.ds(base, L)] = jnp.where(
        v > 0.0, 1.0 / jnp.maximum(v, 1.0), 0.0)
    return 0
  lax.fori_loop(0, R_T // L, _inv, 0)

  for k in range(K_HOPS):
    w = ALPHA ** (k + 1)

    for ch in range(N_RCH):
      row0 = pl.multiple_of(tile_base + ch * 128, 128)
      pltpu.sync_copy(zbuf, acc_sh.at[pl.ds(row0, 128)])
    plsc.subcore_barrier()

    def _edge(b, _):
      bb = pl.multiple_of(b * B_CH, B_CH)
      pltpu.sync_copy(src_r.at[sid, pl.ds(bb, B_CH)], sidxb)
      pltpu.sync_copy(dst_r.at[sid, pl.ds(bb, B_CH)], didxb)
      for jj in range(B_CH):
        pltpu.sync_copy(h_sh.at[sidxb.at[jj]], wbuf0)
        pltpu.sync_copy(wbuf0, acc_sh.at[didxb.at[jj]], add=True)
      return 0
    lax.fori_loop(0, N_BLK, _edge, 0)
    plsc.subcore_barrier()

    for ch in range(N_RCH):
      row0 = pl.multiple_of(tile_base + ch * 128, 128)
      pltpu.sync_copy(acc_sh.at[pl.ds(row0, 128)], wbuf0)
      pltpu.sync_copy(out_hbm.at[cid, pl.ds(row0, 128)], wbuf1)

      def _norm(g, _, ch=ch):
        gbase = pl.multiple_of(g * L, L)
        iv = degbuf[pl.ds(pl.multiple_of(ch * 128, 128) + gbase, L)]
        for r16 in range(L):
          r = gbase + r16
          bc = jnp.full((L,), iv[r16], jnp.float32)
          for f in range(DH // L):
            a = wbuf0[r, pl.ds(f * L, L)]
            h16 = a * bc
            wbuf0[r, pl.ds(f * L, L)] = h16
            wbuf1[r, pl.ds(f * L, L)] = wbuf1[r, pl.ds(f * L, L)] + w * h16
        return 0
      lax.fori_loop(0, 128 // L, _norm, 0)

      if k != K_HOPS - 1:
        pltpu.sync_copy(wbuf0, h_sh.at[pl.ds(row0, 128)])
      pltpu.sync_copy(wbuf1, out_hbm.at[cid, pl.ds(row0, 128)])
    if k != K_HOPS - 1:
      plsc.subcore_barrier()


@jax.jit
def kernel(x, edge_index):
  x_p = jnp.pad(x, ((0, N_PAD - N), (0, 0)))
  x_r = x_p.reshape(N_PAD, NC, DH).transpose(1, 0, 2)

  src = edge_index[0].reshape(NS, E_T)
  dst = edge_index[1].reshape(NS, E_T)
  src_r = jnp.pad(src, ((0, 0), (0, E_T_PAD - E_T)),
                  constant_values=PAD_SRC).reshape(NS, C_T, 128)
  dst_r = jnp.pad(dst, ((0, 0), (0, E_T_PAD - E_T)),
                  constant_values=PAD_DST).reshape(NS, C_T, 128)

  zeros2d = jnp.zeros((128, DH), jnp.float32)
  zeros1d = jnp.zeros((R_T,), jnp.float32)

  mesh = plsc.VectorSubcoreMesh(core_axis_name="c", subcore_axis_name="s",
                                num_cores=NC, num_subcores=NS)
  out_r = pl.kernel(
      _body,
      out_type=jax.ShapeDtypeStruct((NC, N_PAD, DH), jnp.float32),
      mesh=mesh,
      compiler_params=pltpu.CompilerParams(use_tc_tiling_on_sc=False),
      scratch_types=[
          pltpu.VMEM((B_CH, 128), jnp.int32),
          pltpu.VMEM((B_CH, 128), jnp.int32),
          pltpu.VMEM((128, DH), jnp.float32),
          pltpu.VMEM((128, DH), jnp.float32),
          pltpu.VMEM((128, DH), jnp.float32),
          pltpu.VMEM((R_T,), jnp.float32),
          pltpu.VMEM((128,), jnp.float32),
          pltpu.VMEM_SHARED((N_PAD, DH), jnp.float32),
          pltpu.VMEM_SHARED((N_PAD, DH), jnp.float32),
          pltpu.VMEM_SHARED((N_PAD,), jnp.float32),
      ],
  )(x_r, src_r, dst_r, zeros2d, zeros1d)

  out = out_r.transpose(1, 0, 2).reshape(N_PAD, D)
  return out[:N]
